# Initial kernel scaffold; baseline (speedup 1.0000x reference)
import jax
import jax.numpy as jnp
from jax.experimental import pallas as pl


def kernel(atoms, adjacency_map):
    raise NotImplementedError("write your pallas kernel here")


# speedup vs baseline: 3.6224x; 3.6224x over previous
import jax
import jax.numpy as jnp
from jax.experimental import pallas as pl
from jax.experimental.pallas import tpu as pltpu

_N = 2048
_E = 8192
_ROWS = 8
_W = 32


def _row_cumsum(x):
    n = x.shape[-1]
    s = 1
    while s < n:
        shifted = jnp.concatenate([jnp.zeros_like(x[:, :s]), x[:, :-s]], axis=-1)
        x = x + shifted
        s *= 2
    return x


def _compact_kernel(adj_ref, out_ref, off_ref):
    i = pl.program_id(0)

    @pl.when(i == 0)
    def _init():
        off_ref[0] = 0

    block = adj_ref[...]
    mask = (block > 0.0).astype(jnp.float32)
    cum = _row_cumsum(mask)
    kiota = jax.lax.broadcasted_iota(jnp.int32, (_W, _N), 0)

    off = off_ref[0]
    for r in range(_ROWS):
        vrow = block[r, :]
        pos = cum[r, :].astype(jnp.int32) - 1
        hit = (pos[None, :] == kiota) & (vrow[None, :] > 0.0)
        c = jnp.sum(jnp.where(hit, vrow[None, :], 0.0), axis=1)
        out_ref[pl.ds(off, _W), :] = c[:, None]
        cnt = jnp.sum(mask[r, :]).astype(jnp.int32)
        off = off + cnt
    off_ref[0] = off


def kernel(atoms, adjacency_map):
    del atoms
    padded = pl.pallas_call(
        _compact_kernel,
        grid=(_N // _ROWS,),
        in_specs=[pl.BlockSpec((_ROWS, _N), lambda i: (i, 0))],
        out_specs=pl.BlockSpec((_E + _W, 1), lambda i: (0, 0)),
        out_shape=jax.ShapeDtypeStruct((_E + _W, 1), jnp.float32),
        scratch_shapes=[pltpu.SMEM((1,), jnp.int32)],
        compiler_params=pltpu.CompilerParams(
            dimension_semantics=("arbitrary",),
        ),
    )(adjacency_map)
    return padded[:_E]
